# Initial kernel scaffold; baseline (speedup 1.0000x reference)
import jax
import jax.numpy as jnp
from jax.experimental import pallas as pl


def kernel(treatment_node_features, treatment_edges, edge_types, batch_assignments, W_self, W_rel, b, W_out, b_out):
    raise NotImplementedError("write your pallas kernel here")


# speedup vs baseline: 10.9269x; 10.9269x over previous
import functools

import jax
import jax.numpy as jnp
from jax import lax
from jax.experimental import pallas as pl
from jax.experimental.pallas import tpu as pltpu
from jax.experimental.pallas import tpu_sc as plsc

N = 10000
E = 320000
D = 128
R = 4
B = 128
EMB = 128

NC = 2
NS = 16
NW = NC * NS

EC = E // 128
ECP = 2560
CHT = ECP // NW
NSH = 10240
SLICE = NSH // NS

TN = 400
GRID = N // TN

_DOT = functools.partial(lax.dot_general, precision=lax.Precision.HIGHEST,
                         preferred_element_type=jnp.float32)


def _mm(a, w):
    return _DOT(a, w, (((1,), (0,)), ((), ())))


def _xr_body(h_ref, wr_ref, xr_ref):
    h = h_ref[...]
    xr_ref[...] = jnp.stack([_mm(h, wr_ref[r]) for r in range(R)])


def _tc_xr(h, w_rel):
    return pl.pallas_call(
        _xr_body,
        grid=(GRID,),
        in_specs=[
            pl.BlockSpec((TN, D), lambda i: (i, 0)),
            pl.BlockSpec((R, D, D), lambda i: (0, 0, 0)),
        ],
        out_specs=pl.BlockSpec((R, TN, D), lambda i: (0, i, 0)),
        out_shape=jax.ShapeDtypeStruct((R, N, D), jnp.float32),
    )(h, w_rel)


def _mid_body(h_ref, agg_ref, deg_ref, ws_ref, b_ref, wr_ref, h1_ref, xr_ref):
    agg = agg_ref[0] + agg_ref[1]
    deg = deg_ref[0, :, 0:1] + deg_ref[1, :, 0:1]
    deg = jnp.maximum(deg, 1.0)
    h1 = _mm(h_ref[...], ws_ref[...]) + agg / deg + b_ref[...]
    h1 = jnp.maximum(h1, 0.0)
    h1_ref[...] = h1
    xr_ref[...] = jnp.stack([_mm(h1, wr_ref[r]) for r in range(R)])


def _tc_mid(h, agg, deg, w_self, b0, w_rel):
    return pl.pallas_call(
        _mid_body,
        grid=(GRID,),
        in_specs=[
            pl.BlockSpec((TN, D), lambda i: (i, 0)),
            pl.BlockSpec((NC, TN, D), lambda i: (0, i, 0)),
            pl.BlockSpec((NC, TN, D), lambda i: (0, i, 0)),
            pl.BlockSpec((D, D), lambda i: (0, 0)),
            pl.BlockSpec((1, D), lambda i: (0, 0)),
            pl.BlockSpec((R, D, D), lambda i: (0, 0, 0)),
        ],
        out_specs=[
            pl.BlockSpec((TN, D), lambda i: (i, 0)),
            pl.BlockSpec((R, TN, D), lambda i: (0, i, 0)),
        ],
        out_shape=[
            jax.ShapeDtypeStruct((N, D), jnp.float32),
            jax.ShapeDtypeStruct((R, N, D), jnp.float32),
        ],
    )(h, agg, deg, w_self, b0, w_rel)


def _final_body(h1_ref, agg_ref, deg_ref, asg_ref, ws_ref, b_ref, wo_ref,
                bo_ref, out_ref, pool_ref, cnt_ref):
    i = pl.program_id(0)

    @pl.when(i == 0)
    def _():
        pool_ref[...] = jnp.zeros((B, D), jnp.float32)
        cnt_ref[...] = jnp.zeros((8, B), jnp.float32)

    agg = agg_ref[0] + agg_ref[1]
    deg = deg_ref[0, :, 0:1] + deg_ref[1, :, 0:1]
    deg = jnp.maximum(deg, 1.0)
    h2 = _mm(h1_ref[...], ws_ref[...]) + agg / deg + b_ref[...]

    a = asg_ref[0, 0, :]
    onehot = (a[:, None] == lax.broadcasted_iota(jnp.int32, (TN, B), 1))
    onehot = onehot.astype(jnp.float32)
    pool_ref[...] += _DOT(onehot, h2, (((0,), (0,)), ((), ())))
    cnt_ref[0, :] += jnp.sum(onehot, axis=0)

    @pl.when(i == GRID - 1)
    def _():
        cnt = jnp.maximum(cnt_ref[0, :], 1.0)
        pooled = pool_ref[...] / cnt[:, None]
        out_ref[...] = _mm(pooled, wo_ref[...]) + bo_ref[...]


def _tc_final(h1, agg, deg, asg, w_self, b1, w_out, b_out):
    return pl.pallas_call(
        _final_body,
        grid=(GRID,),
        in_specs=[
            pl.BlockSpec((TN, D), lambda i: (i, 0)),
            pl.BlockSpec((NC, TN, D), lambda i: (0, i, 0)),
            pl.BlockSpec((NC, TN, D), lambda i: (0, i, 0)),
            pl.BlockSpec((1, 1, TN), lambda i: (i, 0, 0)),
            pl.BlockSpec((D, D), lambda i: (0, 0)),
            pl.BlockSpec((1, D), lambda i: (0, 0)),
            pl.BlockSpec((D, EMB), lambda i: (0, 0)),
            pl.BlockSpec((1, EMB), lambda i: (0, 0)),
        ],
        out_specs=pl.BlockSpec((B, EMB), lambda i: (0, 0)),
        out_shape=jax.ShapeDtypeStruct((B, EMB), jnp.float32),
        scratch_shapes=[
            pltpu.VMEM((B, D), jnp.float32),
            pltpu.VMEM((8, B), jnp.float32),
        ],
    )(h1, agg, deg, asg, w_self, b1, w_out, b_out)


_MESH = plsc.VectorSubcoreMesh(core_axis_name="c", subcore_axis_name="s",
                               num_cores=NC, num_subcores=NS)

GR = 16
NG = CHT // GR


def _edge_body(xr_hbm, src_hbm, et_hbm, dst_hbm, z_hbm,
               agg_out,
               srcv, etv, dstv, flatv, rows, agg_sh, sem):
    c = lax.axis_index("c")
    s = lax.axis_index("s")
    base = (c * NS + s) * CHT
    row0 = s * SLICE
    pltpu.sync_copy(z_hbm, agg_sh.at[pl.ds(row0, SLICE)])
    plsc.subcore_barrier()

    def gbody(g, carry):
        off = base + g * GR
        pltpu.sync_copy(src_hbm.at[pl.ds(off, GR)], srcv)
        pltpu.sync_copy(et_hbm.at[pl.ds(off, GR)], etv)
        pltpu.sync_copy(dst_hbm.at[pl.ds(off, GR)], dstv)

        def fbody(j, cc):
            for k in range(8):
                sl = pl.ds(k * 16, 16)
                flatv[j, sl] = etv[j, sl] * N + srcv[j, sl]
            return cc

        lax.fori_loop(0, GR, fbody, 0)

        def ebody(j, cc):
            pltpu.async_copy(xr_hbm.at[flatv.at[j]], rows, sem).wait()
            pltpu.sync_copy(rows, agg_sh.at[dstv.at[j]], add=True)
            return cc

        lax.fori_loop(0, GR, ebody, 0)
        return carry

    lax.fori_loop(0, NG, gbody, 0)
    plsc.subcore_barrier()
    pltpu.sync_copy(agg_sh.at[pl.ds(row0, SLICE)],
                    agg_out.at[c, pl.ds(row0, SLICE)])


def _deg_body(dst_hbm, z_hbm, ones_hbm, deg_out, dstv, ones_v, deg_sh):
    c = lax.axis_index("c")
    s = lax.axis_index("s")
    base = (c * NS + s) * CHT
    row0 = s * SLICE
    pltpu.sync_copy(z_hbm, deg_sh.at[pl.ds(row0, SLICE)])
    pltpu.sync_copy(ones_hbm, ones_v)
    plsc.subcore_barrier()

    def gbody(g, carry):
        off = base + g * GR
        pltpu.sync_copy(dst_hbm.at[pl.ds(off, GR)], dstv)

        def ebody(j, cc):
            pltpu.sync_copy(ones_v, deg_sh.at[dstv.at[j]], add=True)
            return cc

        lax.fori_loop(0, GR, ebody, 0)
        return carry

    lax.fori_loop(0, NG, gbody, 0)
    plsc.subcore_barrier()
    pltpu.sync_copy(deg_sh.at[pl.ds(row0, SLICE)],
                    deg_out.at[c, pl.ds(row0, SLICE)])


_sc_edge = pl.kernel(
    _edge_body,
    out_type=jax.ShapeDtypeStruct((NC, NSH, D), jnp.float32),
    mesh=_MESH,
    scratch_types=[
        pltpu.VMEM((GR, 128), jnp.int32),
        pltpu.VMEM((GR, 128), jnp.int32),
        pltpu.VMEM((GR, 128), jnp.int32),
        pltpu.VMEM((GR, 128), jnp.int32),
        pltpu.VMEM((128, D), jnp.float32),
        pltpu.VMEM_SHARED((NSH, D), jnp.float32),
        pltpu.SemaphoreType.DMA,
    ],
)

_sc_deg = pl.kernel(
    _deg_body,
    out_type=jax.ShapeDtypeStruct((NC, NSH, D), jnp.float32),
    mesh=_MESH,
    scratch_types=[
        pltpu.VMEM((GR, 128), jnp.int32),
        pltpu.VMEM((128, D), jnp.float32),
        pltpu.VMEM_SHARED((NSH, D), jnp.float32),
    ],
)


def kernel(treatment_node_features, treatment_edges, edge_types,
           batch_assignments, W_self, W_rel, b, W_out, b_out):
    h = treatment_node_features.astype(jnp.float32)
    src = treatment_edges[0].astype(jnp.int32)
    dst = treatment_edges[1].astype(jnp.int32)
    et = edge_types.astype(jnp.int32)

    pad = ECP * 128 - E
    srcp = jnp.concatenate([src, jnp.zeros((pad,), jnp.int32)]).reshape(ECP, 128)
    etp = jnp.concatenate([et, jnp.zeros((pad,), jnp.int32)]).reshape(ECP, 128)
    dstp = jnp.concatenate([dst, jnp.full((pad,), N, jnp.int32)]).reshape(ECP, 128)

    z_hbm = jnp.zeros((SLICE, D), jnp.float32)
    ones_hbm = jnp.ones((128, D), jnp.float32)

    degp = _sc_deg(dstp, z_hbm, ones_hbm)
    xr1 = _tc_xr(h, W_rel[0])
    agg1 = _sc_edge(xr1.reshape(R * N, D), srcp, etp, dstp, z_hbm)
    h1, xr2 = _tc_mid(h, agg1, degp, W_self[0], b[0].reshape(1, D), W_rel[1])
    agg2 = _sc_edge(xr2.reshape(R * N, D), srcp, etp, dstp, z_hbm)

    asg = batch_assignments.astype(jnp.int32).reshape(GRID, 1, TN)
    out = _tc_final(h1, agg2, degp, asg, W_self[1], b[1].reshape(1, D),
                    W_out, b_out.reshape(1, EMB))
    return out[:, None, :]


# speedup vs baseline: 12.5120x; 1.1451x over previous
import functools

import jax
import jax.numpy as jnp
from jax import lax
from jax.experimental import pallas as pl
from jax.experimental.pallas import tpu as pltpu
from jax.experimental.pallas import tpu_sc as plsc

N = 10000
E = 320000
D = 128
R = 4
B = 128
EMB = 128

NC = 2
NS = 16
NW = NC * NS

EC = E // 128
ECP = 2560
CHT = ECP // NW
NSH = 10240
SLICE = NSH // NS

TN = 400
GRID = N // TN

_DOT = functools.partial(lax.dot_general, precision=lax.Precision.HIGHEST,
                         preferred_element_type=jnp.float32)


def _mm(a, w):
    return _DOT(a, w, (((1,), (0,)), ((), ())))


def _xr_body(h_ref, wr_ref, xr_ref):
    h = h_ref[...]
    xr_ref[...] = jnp.stack([_mm(h, wr_ref[r]) for r in range(R)])


def _tc_xr(h, w_rel):
    return pl.pallas_call(
        _xr_body,
        grid=(GRID,),
        in_specs=[
            pl.BlockSpec((TN, D), lambda i: (i, 0)),
            pl.BlockSpec((R, D, D), lambda i: (0, 0, 0)),
        ],
        out_specs=pl.BlockSpec((R, TN, D), lambda i: (0, i, 0)),
        out_shape=jax.ShapeDtypeStruct((R, N, D), jnp.float32),
    )(h, w_rel)


def _mid_body(h_ref, agg_ref, deg_ref, ws_ref, b_ref, wr_ref, h1_ref, xr_ref):
    agg = agg_ref[0] + agg_ref[1]
    deg = deg_ref[0, :, 0:1] + deg_ref[1, :, 0:1]
    deg = jnp.maximum(deg, 1.0)
    h1 = _mm(h_ref[...], ws_ref[...]) + agg / deg + b_ref[...]
    h1 = jnp.maximum(h1, 0.0)
    h1_ref[...] = h1
    xr_ref[...] = jnp.stack([_mm(h1, wr_ref[r]) for r in range(R)])


def _tc_mid(h, agg, deg, w_self, b0, w_rel):
    return pl.pallas_call(
        _mid_body,
        grid=(GRID,),
        in_specs=[
            pl.BlockSpec((TN, D), lambda i: (i, 0)),
            pl.BlockSpec((NC, TN, D), lambda i: (0, i, 0)),
            pl.BlockSpec((NC, TN, D), lambda i: (0, i, 0)),
            pl.BlockSpec((D, D), lambda i: (0, 0)),
            pl.BlockSpec((1, D), lambda i: (0, 0)),
            pl.BlockSpec((R, D, D), lambda i: (0, 0, 0)),
        ],
        out_specs=[
            pl.BlockSpec((TN, D), lambda i: (i, 0)),
            pl.BlockSpec((R, TN, D), lambda i: (0, i, 0)),
        ],
        out_shape=[
            jax.ShapeDtypeStruct((N, D), jnp.float32),
            jax.ShapeDtypeStruct((R, N, D), jnp.float32),
        ],
    )(h, agg, deg, w_self, b0, w_rel)


def _final_body(h1_ref, agg_ref, deg_ref, asg_ref, ws_ref, b_ref, wo_ref,
                bo_ref, out_ref, pool_ref, cnt_ref):
    i = pl.program_id(0)

    @pl.when(i == 0)
    def _():
        pool_ref[...] = jnp.zeros((B, D), jnp.float32)
        cnt_ref[...] = jnp.zeros((8, B), jnp.float32)

    agg = agg_ref[0] + agg_ref[1]
    deg = deg_ref[0, :, 0:1] + deg_ref[1, :, 0:1]
    deg = jnp.maximum(deg, 1.0)
    h2 = _mm(h1_ref[...], ws_ref[...]) + agg / deg + b_ref[...]

    a = asg_ref[0, 0, :]
    onehot = (a[:, None] == lax.broadcasted_iota(jnp.int32, (TN, B), 1))
    onehot = onehot.astype(jnp.float32)
    pool_ref[...] += _DOT(onehot, h2, (((0,), (0,)), ((), ())))
    cnt_ref[0, :] += jnp.sum(onehot, axis=0)

    @pl.when(i == GRID - 1)
    def _():
        cnt = jnp.maximum(cnt_ref[0, :], 1.0)
        pooled = pool_ref[...] / cnt[:, None]
        out_ref[...] = _mm(pooled, wo_ref[...]) + bo_ref[...]


def _tc_final(h1, agg, deg, asg, w_self, b1, w_out, b_out):
    return pl.pallas_call(
        _final_body,
        grid=(GRID,),
        in_specs=[
            pl.BlockSpec((TN, D), lambda i: (i, 0)),
            pl.BlockSpec((NC, TN, D), lambda i: (0, i, 0)),
            pl.BlockSpec((NC, TN, D), lambda i: (0, i, 0)),
            pl.BlockSpec((1, 1, TN), lambda i: (i, 0, 0)),
            pl.BlockSpec((D, D), lambda i: (0, 0)),
            pl.BlockSpec((1, D), lambda i: (0, 0)),
            pl.BlockSpec((D, EMB), lambda i: (0, 0)),
            pl.BlockSpec((1, EMB), lambda i: (0, 0)),
        ],
        out_specs=pl.BlockSpec((B, EMB), lambda i: (0, 0)),
        out_shape=jax.ShapeDtypeStruct((B, EMB), jnp.float32),
        scratch_shapes=[
            pltpu.VMEM((B, D), jnp.float32),
            pltpu.VMEM((8, B), jnp.float32),
        ],
    )(h1, agg, deg, asg, w_self, b1, w_out, b_out)


_MESH = plsc.VectorSubcoreMesh(core_axis_name="c", subcore_axis_name="s",
                               num_cores=NC, num_subcores=NS)

GR = 16
NG = CHT // GR


def _edge_body(xr_hbm, flat_hbm, dst_hbm, z_hbm,
               agg_out,
               flatv, dstv, rows0, rows1, agg_sh, sem0, sem1):
    c = lax.axis_index("c")
    s = lax.axis_index("s")
    base = (c * NS + s) * CHT
    row0 = s * SLICE
    pltpu.sync_copy(z_hbm, agg_sh.at[pl.ds(row0, SLICE)])
    plsc.subcore_barrier()

    def gbody(g, carry):
        off = base + g * GR
        pltpu.sync_copy(flat_hbm.at[pl.ds(off, GR)], flatv)
        pltpu.sync_copy(dst_hbm.at[pl.ds(off, GR)], dstv)
        pltpu.async_copy(xr_hbm.at[flatv.at[0]], rows0, sem0)

        def ebody(i, cc):
            j0 = 2 * i
            pltpu.async_copy(xr_hbm.at[flatv.at[j0 + 1]], rows1, sem1)
            pltpu.make_async_copy(xr_hbm.at[flatv.at[j0]], rows0, sem0).wait()
            pltpu.sync_copy(rows0, agg_sh.at[dstv.at[j0]], add=True)
            pltpu.async_copy(xr_hbm.at[flatv.at[j0 + 2]], rows0, sem0)
            pltpu.make_async_copy(xr_hbm.at[flatv.at[j0 + 1]], rows1, sem1).wait()
            pltpu.sync_copy(rows1, agg_sh.at[dstv.at[j0 + 1]], add=True)
            return cc

        lax.fori_loop(0, GR // 2 - 1, ebody, 0)

        pltpu.async_copy(xr_hbm.at[flatv.at[GR - 1]], rows1, sem1)
        pltpu.make_async_copy(xr_hbm.at[flatv.at[GR - 2]], rows0, sem0).wait()
        pltpu.sync_copy(rows0, agg_sh.at[dstv.at[GR - 2]], add=True)
        pltpu.make_async_copy(xr_hbm.at[flatv.at[GR - 1]], rows1, sem1).wait()
        pltpu.sync_copy(rows1, agg_sh.at[dstv.at[GR - 1]], add=True)
        return carry

    lax.fori_loop(0, NG, gbody, 0)
    plsc.subcore_barrier()
    pltpu.sync_copy(agg_sh.at[pl.ds(row0, SLICE)],
                    agg_out.at[c, pl.ds(row0, SLICE)])


def _deg_body(dst_hbm, z_hbm, ones_hbm, deg_out, dstv, ones_v, deg_sh):
    c = lax.axis_index("c")
    s = lax.axis_index("s")
    base = (c * NS + s) * CHT
    row0 = s * SLICE
    pltpu.sync_copy(z_hbm, deg_sh.at[pl.ds(row0, SLICE)])
    pltpu.sync_copy(ones_hbm, ones_v)
    plsc.subcore_barrier()

    def gbody(g, carry):
        off = base + g * GR
        pltpu.sync_copy(dst_hbm.at[pl.ds(off, GR)], dstv)

        def ebody(j, cc):
            pltpu.sync_copy(ones_v, deg_sh.at[dstv.at[j]], add=True)
            return cc

        lax.fori_loop(0, GR, ebody, 0)
        return carry

    lax.fori_loop(0, NG, gbody, 0)
    plsc.subcore_barrier()
    pltpu.sync_copy(deg_sh.at[pl.ds(row0, SLICE)],
                    deg_out.at[c, pl.ds(row0, SLICE)])


_sc_edge = pl.kernel(
    _edge_body,
    out_type=jax.ShapeDtypeStruct((NC, NSH, D), jnp.float32),
    mesh=_MESH,
    scratch_types=[
        pltpu.VMEM((GR, 128), jnp.int32),
        pltpu.VMEM((GR, 128), jnp.int32),
        pltpu.VMEM((128, D), jnp.float32),
        pltpu.VMEM((128, D), jnp.float32),
        pltpu.VMEM_SHARED((NSH, D), jnp.float32),
        pltpu.SemaphoreType.DMA,
        pltpu.SemaphoreType.DMA,
    ],
)

_sc_deg = pl.kernel(
    _deg_body,
    out_type=jax.ShapeDtypeStruct((NC, NSH, D), jnp.float32),
    mesh=_MESH,
    scratch_types=[
        pltpu.VMEM((GR, 128), jnp.int32),
        pltpu.VMEM((128, D), jnp.float32),
        pltpu.VMEM_SHARED((NSH, D), jnp.float32),
    ],
)


def kernel(treatment_node_features, treatment_edges, edge_types,
           batch_assignments, W_self, W_rel, b, W_out, b_out):
    h = treatment_node_features.astype(jnp.float32)
    src = treatment_edges[0].astype(jnp.int32)
    dst = treatment_edges[1].astype(jnp.int32)
    et = edge_types.astype(jnp.int32)

    pad = ECP * 128 - E
    flat = et * N + src
    flatp = jnp.concatenate([flat, jnp.zeros((pad,), jnp.int32)]).reshape(ECP, 128)
    dstp = jnp.concatenate([dst, jnp.full((pad,), N, jnp.int32)]).reshape(ECP, 128)

    z_hbm = jnp.zeros((SLICE, D), jnp.float32)
    ones_hbm = jnp.ones((128, D), jnp.float32)

    degp = _sc_deg(dstp, z_hbm, ones_hbm)
    xr1 = _tc_xr(h, W_rel[0])
    agg1 = _sc_edge(xr1.reshape(R * N, D), flatp, dstp, z_hbm)
    h1, xr2 = _tc_mid(h, agg1, degp, W_self[0], b[0].reshape(1, D), W_rel[1])
    agg2 = _sc_edge(xr2.reshape(R * N, D), flatp, dstp, z_hbm)

    asg = batch_assignments.astype(jnp.int32).reshape(GRID, 1, TN)
    out = _tc_final(h1, agg2, degp, asg, W_self[1], b[1].reshape(1, D),
                    W_out, b_out.reshape(1, EMB))
    return out[:, None, :]


# speedup vs baseline: 27.7422x; 2.2172x over previous
import functools

import jax
import jax.numpy as jnp
from jax import lax
from jax.experimental import pallas as pl
from jax.experimental.pallas import tpu as pltpu
from jax.experimental.pallas import tpu_sc as plsc

N = 10000
E = 320000
D = 128
R = 4
B = 128
EMB = 128

NC = 2
NS = 16
NW = NC * NS

EC = E // 128
ECP = 2560
CHT = ECP // NW
NSH = 10240
SLICE = NSH // NS

TN = 400
GRID = N // TN

_DOT = functools.partial(lax.dot_general, precision=lax.Precision.HIGHEST,
                         preferred_element_type=jnp.float32)


def _mm(a, w):
    return _DOT(a, w, (((1,), (0,)), ((), ())))


def _xr_body(h_ref, wr_ref, xr_ref):
    h = h_ref[...]
    xr_ref[...] = jnp.stack([_mm(h, wr_ref[r]) for r in range(R)])


def _tc_xr(h, w_rel):
    return pl.pallas_call(
        _xr_body,
        grid=(GRID,),
        in_specs=[
            pl.BlockSpec((TN, D), lambda i: (i, 0)),
            pl.BlockSpec((R, D, D), lambda i: (0, 0, 0)),
        ],
        out_specs=pl.BlockSpec((R, TN, D), lambda i: (0, i, 0)),
        out_shape=jax.ShapeDtypeStruct((R, N, D), jnp.float32),
    )(h, w_rel)


def _mid_body(h_ref, agg_ref, deg_ref, ws_ref, b_ref, wr_ref, h1_ref, xr_ref):
    agg = agg_ref[0] + agg_ref[1]
    deg = deg_ref[0, :, 0:1] + deg_ref[1, :, 0:1]
    deg = jnp.maximum(deg, 1.0)
    h1 = _mm(h_ref[...], ws_ref[...]) + agg / deg + b_ref[...]
    h1 = jnp.maximum(h1, 0.0)
    h1_ref[...] = h1
    xr_ref[...] = jnp.stack([_mm(h1, wr_ref[r]) for r in range(R)])


def _tc_mid(h, agg, deg, w_self, b0, w_rel):
    return pl.pallas_call(
        _mid_body,
        grid=(GRID,),
        in_specs=[
            pl.BlockSpec((TN, D), lambda i: (i, 0)),
            pl.BlockSpec((NC, TN, D), lambda i: (0, i, 0)),
            pl.BlockSpec((NC, TN, D), lambda i: (0, i, 0)),
            pl.BlockSpec((D, D), lambda i: (0, 0)),
            pl.BlockSpec((1, D), lambda i: (0, 0)),
            pl.BlockSpec((R, D, D), lambda i: (0, 0, 0)),
        ],
        out_specs=[
            pl.BlockSpec((TN, D), lambda i: (i, 0)),
            pl.BlockSpec((R, TN, D), lambda i: (0, i, 0)),
        ],
        out_shape=[
            jax.ShapeDtypeStruct((N, D), jnp.float32),
            jax.ShapeDtypeStruct((R, N, D), jnp.float32),
        ],
    )(h, agg, deg, w_self, b0, w_rel)


def _final_body(h1_ref, agg_ref, deg_ref, asg_ref, ws_ref, b_ref, wo_ref,
                bo_ref, out_ref, pool_ref, cnt_ref):
    i = pl.program_id(0)

    @pl.when(i == 0)
    def _():
        pool_ref[...] = jnp.zeros((B, D), jnp.float32)
        cnt_ref[...] = jnp.zeros((8, B), jnp.float32)

    agg = agg_ref[0] + agg_ref[1]
    deg = deg_ref[0, :, 0:1] + deg_ref[1, :, 0:1]
    deg = jnp.maximum(deg, 1.0)
    h2 = _mm(h1_ref[...], ws_ref[...]) + agg / deg + b_ref[...]

    a = asg_ref[0, 0, :]
    onehot = (a[:, None] == lax.broadcasted_iota(jnp.int32, (TN, B), 1))
    onehot = onehot.astype(jnp.float32)
    pool_ref[...] += _DOT(onehot, h2, (((0,), (0,)), ((), ())))
    cnt_ref[0, :] += jnp.sum(onehot, axis=0)

    @pl.when(i == GRID - 1)
    def _():
        cnt = jnp.maximum(cnt_ref[0, :], 1.0)
        pooled = pool_ref[...] / cnt[:, None]
        out_ref[...] = _mm(pooled, wo_ref[...]) + bo_ref[...]


def _tc_final(h1, agg, deg, asg, w_self, b1, w_out, b_out):
    return pl.pallas_call(
        _final_body,
        grid=(GRID,),
        in_specs=[
            pl.BlockSpec((TN, D), lambda i: (i, 0)),
            pl.BlockSpec((NC, TN, D), lambda i: (0, i, 0)),
            pl.BlockSpec((NC, TN, D), lambda i: (0, i, 0)),
            pl.BlockSpec((1, 1, TN), lambda i: (i, 0, 0)),
            pl.BlockSpec((D, D), lambda i: (0, 0)),
            pl.BlockSpec((1, D), lambda i: (0, 0)),
            pl.BlockSpec((D, EMB), lambda i: (0, 0)),
            pl.BlockSpec((1, EMB), lambda i: (0, 0)),
        ],
        out_specs=pl.BlockSpec((B, EMB), lambda i: (0, 0)),
        out_shape=jax.ShapeDtypeStruct((B, EMB), jnp.float32),
        scratch_shapes=[
            pltpu.VMEM((B, D), jnp.float32),
            pltpu.VMEM((8, B), jnp.float32),
        ],
    )(h1, agg, deg, asg, w_self, b1, w_out, b_out)


_MESH = plsc.VectorSubcoreMesh(core_axis_name="c", subcore_axis_name="s",
                               num_cores=NC, num_subcores=NS)

GR = 16
NG = CHT // GR


def _edge_body(xr_hbm, flat_hbm, dst_hbm, z_hbm,
               agg_out,
               flatv, dstv, rows0, rows1, agg_sh, sem0, sem1):
    c = lax.axis_index("c")
    s = lax.axis_index("s")
    base = (c * NS + s) * CHT
    row0 = s * SLICE
    pltpu.sync_copy(z_hbm, agg_sh.at[pl.ds(row0, SLICE)])
    plsc.subcore_barrier()

    def gbody(g, carry):
        off = base + g * GR
        pltpu.sync_copy(flat_hbm.at[pl.ds(off, GR)], flatv)
        pltpu.sync_copy(dst_hbm.at[pl.ds(off, GR)], dstv)
        pltpu.async_copy(xr_hbm.at[flatv.at[0]], rows0, sem0)

        def ebody(i, cc):
            j0 = 2 * i
            pltpu.async_copy(xr_hbm.at[flatv.at[j0 + 1]], rows1, sem1)
            pltpu.make_async_copy(xr_hbm.at[flatv.at[j0]], rows0, sem0).wait()
            pltpu.sync_copy(rows0, agg_sh.at[dstv.at[j0]], add=True)
            pltpu.async_copy(xr_hbm.at[flatv.at[j0 + 2]], rows0, sem0)
            pltpu.make_async_copy(xr_hbm.at[flatv.at[j0 + 1]], rows1, sem1).wait()
            pltpu.sync_copy(rows1, agg_sh.at[dstv.at[j0 + 1]], add=True)
            return cc

        lax.fori_loop(0, GR // 2 - 1, ebody, 0)

        pltpu.async_copy(xr_hbm.at[flatv.at[GR - 1]], rows1, sem1)
        pltpu.make_async_copy(xr_hbm.at[flatv.at[GR - 2]], rows0, sem0).wait()
        pltpu.sync_copy(rows0, agg_sh.at[dstv.at[GR - 2]], add=True)
        pltpu.make_async_copy(xr_hbm.at[flatv.at[GR - 1]], rows1, sem1).wait()
        pltpu.sync_copy(rows1, agg_sh.at[dstv.at[GR - 1]], add=True)
        return carry

    lax.fori_loop(0, NG, gbody, 0)
    plsc.subcore_barrier()
    pltpu.sync_copy(agg_sh.at[pl.ds(row0, SLICE)],
                    agg_out.at[c, pl.ds(row0, SLICE)])


def _deg_body(dst_hbm, z_hbm, ones_hbm, deg_out, dstv, ones_v, deg_sh):
    c = lax.axis_index("c")
    s = lax.axis_index("s")
    base = (c * NS + s) * CHT
    row0 = s * SLICE
    pltpu.sync_copy(z_hbm, deg_sh.at[pl.ds(row0, SLICE)])
    pltpu.sync_copy(ones_hbm, ones_v)
    plsc.subcore_barrier()

    def gbody(g, carry):
        off = base + g * GR
        pltpu.sync_copy(dst_hbm.at[pl.ds(off, GR)], dstv)

        def ebody(j, cc):
            pltpu.sync_copy(ones_v, deg_sh.at[dstv.at[j]], add=True)
            return cc

        lax.fori_loop(0, GR, ebody, 0)
        return carry

    lax.fori_loop(0, NG, gbody, 0)
    plsc.subcore_barrier()
    pltpu.sync_copy(deg_sh.at[pl.ds(row0, SLICE)],
                    deg_out.at[c, pl.ds(row0, SLICE)])


_sc_edge = pl.kernel(
    _edge_body,
    out_type=jax.ShapeDtypeStruct((NC, NSH, D), jnp.float32),
    mesh=_MESH,
    scratch_types=[
        pltpu.VMEM((GR, 128), jnp.int32),
        pltpu.VMEM((GR, 128), jnp.int32),
        pltpu.VMEM((128, D), jnp.float32),
        pltpu.VMEM((128, D), jnp.float32),
        pltpu.VMEM_SHARED((NSH, D), jnp.float32),
        pltpu.SemaphoreType.DMA,
        pltpu.SemaphoreType.DMA,
    ],
)

_sc_deg = pl.kernel(
    _deg_body,
    out_type=jax.ShapeDtypeStruct((NC, NSH, D), jnp.float32),
    mesh=_MESH,
    scratch_types=[
        pltpu.VMEM((GR, 128), jnp.int32),
        pltpu.VMEM((128, D), jnp.float32),
        pltpu.VMEM_SHARED((NSH, D), jnp.float32),
    ],
)


def kernel(treatment_node_features, treatment_edges, edge_types,
           batch_assignments, W_self, W_rel, b, W_out, b_out):
    h = treatment_node_features.astype(jnp.float32)
    src = treatment_edges[0].astype(jnp.int32)
    dst = treatment_edges[1].astype(jnp.int32)
    et = edge_types.astype(jnp.int32)

    pad = ECP * 128 - E
    padi = jnp.arange(pad, dtype=jnp.int32)
    flat = et * N + src
    flatp = jnp.concatenate([flat, padi % (R * N)]).reshape(ECP, 128)
    dstp = jnp.concatenate([dst, N + padi % (NSH - N)]).reshape(ECP, 128)

    z_hbm = jnp.zeros((SLICE, D), jnp.float32)
    ones_hbm = jnp.ones((128, D), jnp.float32)

    degp = _sc_deg(dstp, z_hbm, ones_hbm)
    xr1 = _tc_xr(h, W_rel[0])
    agg1 = _sc_edge(xr1.reshape(R * N, D), flatp, dstp, z_hbm)
    h1, xr2 = _tc_mid(h, agg1, degp, W_self[0], b[0].reshape(1, D), W_rel[1])
    agg2 = _sc_edge(xr2.reshape(R * N, D), flatp, dstp, z_hbm)

    asg = batch_assignments.astype(jnp.int32).reshape(GRID, 1, TN)
    out = _tc_final(h1, agg2, degp, asg, W_self[1], b[1].reshape(1, D),
                    W_out, b_out.reshape(1, EMB))
    return out[:, None, :]


# speedup vs baseline: 27.9066x; 1.0059x over previous
import functools

import jax
import jax.numpy as jnp
from jax import lax
from jax.experimental import pallas as pl
from jax.experimental.pallas import tpu as pltpu
from jax.experimental.pallas import tpu_sc as plsc

N = 10000
E = 320000
D = 128
R = 4
B = 128
EMB = 128

NC = 2
NS = 16
NW = NC * NS

EC = E // 128
ECP = 2560
CHT = ECP // NW
NSH = 10240
SLICE = NSH // NS

TN = 400
GRID = N // TN

_DOT = functools.partial(lax.dot_general, precision=lax.Precision.HIGHEST,
                         preferred_element_type=jnp.float32)


def _mm(a, w):
    return _DOT(a, w, (((1,), (0,)), ((), ())))


def _xr_body(h_ref, wr_ref, xr_ref):
    h = h_ref[...]
    xr_ref[...] = jnp.stack([_mm(h, wr_ref[r]) for r in range(R)])


def _tc_xr(h, w_rel):
    return pl.pallas_call(
        _xr_body,
        grid=(GRID,),
        in_specs=[
            pl.BlockSpec((TN, D), lambda i: (i, 0)),
            pl.BlockSpec((R, D, D), lambda i: (0, 0, 0)),
        ],
        out_specs=pl.BlockSpec((R, TN, D), lambda i: (0, i, 0)),
        out_shape=jax.ShapeDtypeStruct((R, N, D), jnp.float32),
    )(h, w_rel)


def _mid_body(h_ref, agg_ref, deg_ref, ws_ref, b_ref, wr_ref, h1_ref, xr_ref):
    agg = agg_ref[0] + agg_ref[1]
    deg = deg_ref[0, :, 0:1] + deg_ref[1, :, 0:1]
    deg = jnp.maximum(deg, 1.0)
    h1 = _mm(h_ref[...], ws_ref[...]) + agg / deg + b_ref[...]
    h1 = jnp.maximum(h1, 0.0)
    h1_ref[...] = h1
    xr_ref[...] = jnp.stack([_mm(h1, wr_ref[r]) for r in range(R)])


def _tc_mid(h, agg, deg, w_self, b0, w_rel):
    return pl.pallas_call(
        _mid_body,
        grid=(GRID,),
        in_specs=[
            pl.BlockSpec((TN, D), lambda i: (i, 0)),
            pl.BlockSpec((NC, TN, D), lambda i: (0, i, 0)),
            pl.BlockSpec((NC, TN, D), lambda i: (0, i, 0)),
            pl.BlockSpec((D, D), lambda i: (0, 0)),
            pl.BlockSpec((1, D), lambda i: (0, 0)),
            pl.BlockSpec((R, D, D), lambda i: (0, 0, 0)),
        ],
        out_specs=[
            pl.BlockSpec((TN, D), lambda i: (i, 0)),
            pl.BlockSpec((R, TN, D), lambda i: (0, i, 0)),
        ],
        out_shape=[
            jax.ShapeDtypeStruct((N, D), jnp.float32),
            jax.ShapeDtypeStruct((R, N, D), jnp.float32),
        ],
    )(h, agg, deg, w_self, b0, w_rel)


def _final_body(h1_ref, agg_ref, deg_ref, asg_ref, ws_ref, b_ref, wo_ref,
                bo_ref, out_ref, pool_ref, cnt_ref):
    i = pl.program_id(0)

    @pl.when(i == 0)
    def _():
        pool_ref[...] = jnp.zeros((B, D), jnp.float32)
        cnt_ref[...] = jnp.zeros((8, B), jnp.float32)

    agg = agg_ref[0] + agg_ref[1]
    deg = deg_ref[0, :, 0:1] + deg_ref[1, :, 0:1]
    deg = jnp.maximum(deg, 1.0)
    h2 = _mm(h1_ref[...], ws_ref[...]) + agg / deg + b_ref[...]

    a = asg_ref[0, 0, :]
    onehot = (a[:, None] == lax.broadcasted_iota(jnp.int32, (TN, B), 1))
    onehot = onehot.astype(jnp.float32)
    pool_ref[...] += _DOT(onehot, h2, (((0,), (0,)), ((), ())))
    cnt_ref[0, :] += jnp.sum(onehot, axis=0)

    @pl.when(i == GRID - 1)
    def _():
        cnt = jnp.maximum(cnt_ref[0, :], 1.0)
        pooled = pool_ref[...] / cnt[:, None]
        out_ref[...] = _mm(pooled, wo_ref[...]) + bo_ref[...]


def _tc_final(h1, agg, deg, asg, w_self, b1, w_out, b_out):
    return pl.pallas_call(
        _final_body,
        grid=(GRID,),
        in_specs=[
            pl.BlockSpec((TN, D), lambda i: (i, 0)),
            pl.BlockSpec((NC, TN, D), lambda i: (0, i, 0)),
            pl.BlockSpec((NC, TN, D), lambda i: (0, i, 0)),
            pl.BlockSpec((1, 1, TN), lambda i: (i, 0, 0)),
            pl.BlockSpec((D, D), lambda i: (0, 0)),
            pl.BlockSpec((1, D), lambda i: (0, 0)),
            pl.BlockSpec((D, EMB), lambda i: (0, 0)),
            pl.BlockSpec((1, EMB), lambda i: (0, 0)),
        ],
        out_specs=pl.BlockSpec((B, EMB), lambda i: (0, 0)),
        out_shape=jax.ShapeDtypeStruct((B, EMB), jnp.float32),
        scratch_shapes=[
            pltpu.VMEM((B, D), jnp.float32),
            pltpu.VMEM((8, B), jnp.float32),
        ],
    )(h1, agg, deg, asg, w_self, b1, w_out, b_out)


_MESH = plsc.VectorSubcoreMesh(core_axis_name="c", subcore_axis_name="s",
                               num_cores=NC, num_subcores=NS)

NB = 2
GRP = 16


def _edge_body(xr_hbm, flat_hbm, dst_hbm, z_hbm,
               agg_out,
               flatv, dstv, rows, agg_sh, sem0, sem1):
    c = lax.axis_index("c")
    s = lax.axis_index("s")
    base = (c * NS + s) * CHT
    row0 = s * SLICE
    sems = [sem0, sem1]

    pltpu.sync_copy(z_hbm, agg_sh.at[pl.ds(row0, SLICE)])
    plsc.subcore_barrier()

    def gather(j, k):
        pltpu.async_copy(xr_hbm.at[flatv.at[j]],
                         rows.at[pl.ds(k * 128, 128)], sems[k])

    def drain(j, k):
        pltpu.make_async_copy(xr_hbm.at[flatv.at[j]],
                              rows.at[pl.ds(k * 128, 128)], sems[k]).wait()
        pltpu.sync_copy(rows.at[pl.ds(k * 128, 128)],
                        agg_sh.at[dstv.at[j]], add=True)

    def group(g, carry):
        pltpu.sync_copy(flat_hbm.at[pl.ds(base + g * GRP, GRP)], flatv)
        pltpu.sync_copy(dst_hbm.at[pl.ds(base + g * GRP, GRP)], dstv)
        for k in range(NB):
            gather(k, k)
        for j in range(GRP - NB):
            drain(j, j % NB)
            gather(j + NB, j % NB)
        for j in range(GRP - NB, GRP):
            drain(j, j % NB)
        return carry

    lax.fori_loop(0, CHT // GRP, group, 0)

    plsc.subcore_barrier()
    pltpu.sync_copy(agg_sh.at[pl.ds(row0, SLICE)],
                    agg_out.at[c, pl.ds(row0, SLICE)])


def _deg_body(dst_hbm, z_hbm, ones_hbm, deg_out, dstv, ones_v, deg_sh):
    c = lax.axis_index("c")
    s = lax.axis_index("s")
    base = (c * NS + s) * CHT
    row0 = s * SLICE
    pltpu.sync_copy(z_hbm, deg_sh.at[pl.ds(row0, SLICE)])
    pltpu.sync_copy(ones_hbm, ones_v)
    pltpu.sync_copy(dst_hbm.at[pl.ds(base, CHT)], dstv)
    plsc.subcore_barrier()

    def ebody(j, cc):
        pltpu.sync_copy(ones_v, deg_sh.at[dstv.at[j]], add=True)
        return cc

    lax.fori_loop(0, CHT, ebody, 0)
    plsc.subcore_barrier()
    pltpu.sync_copy(deg_sh.at[pl.ds(row0, SLICE)],
                    deg_out.at[c, pl.ds(row0, SLICE)])


_sc_edge = pl.kernel(
    _edge_body,
    out_type=jax.ShapeDtypeStruct((NC, NSH, D), jnp.float32),
    mesh=_MESH,
    scratch_types=[
        pltpu.VMEM((GRP, 128), jnp.int32),
        pltpu.VMEM((GRP, 128), jnp.int32),
        pltpu.VMEM((NB * 128, D), jnp.float32),
        pltpu.VMEM_SHARED((NSH, D), jnp.float32),
        pltpu.SemaphoreType.DMA,
        pltpu.SemaphoreType.DMA,
    ],
)

_sc_deg = pl.kernel(
    _deg_body,
    out_type=jax.ShapeDtypeStruct((NC, NSH, D), jnp.float32),
    mesh=_MESH,
    scratch_types=[
        pltpu.VMEM((CHT, 128), jnp.int32),
        pltpu.VMEM((128, D), jnp.float32),
        pltpu.VMEM_SHARED((NSH, D), jnp.float32),
    ],
)


def kernel(treatment_node_features, treatment_edges, edge_types,
           batch_assignments, W_self, W_rel, b, W_out, b_out):
    h = treatment_node_features.astype(jnp.float32)
    src = treatment_edges[0].astype(jnp.int32)
    dst = treatment_edges[1].astype(jnp.int32)
    et = edge_types.astype(jnp.int32)

    pad = ECP * 128 - E
    padi = jnp.arange(pad, dtype=jnp.int32)
    flat = et * N + src
    flatp = jnp.concatenate([flat, padi % (R * N)]).reshape(ECP, 128)
    dstp = jnp.concatenate([dst, N + padi % (NSH - N)]).reshape(ECP, 128)

    z_hbm = jnp.zeros((SLICE, D), jnp.float32)
    ones_hbm = jnp.ones((128, D), jnp.float32)

    degp = _sc_deg(dstp, z_hbm, ones_hbm)
    xr1 = _tc_xr(h, W_rel[0])
    agg1 = _sc_edge(xr1.reshape(R * N, D), flatp, dstp, z_hbm)
    h1, xr2 = _tc_mid(h, agg1, degp, W_self[0], b[0].reshape(1, D), W_rel[1])
    agg2 = _sc_edge(xr2.reshape(R * N, D), flatp, dstp, z_hbm)

    asg = batch_assignments.astype(jnp.int32).reshape(GRID, 1, TN)
    out = _tc_final(h1, agg2, degp, asg, W_self[1], b[1].reshape(1, D),
                    W_out, b_out.reshape(1, EMB))
    return out[:, None, :]
